# baseline (speedup 1.0000x reference)
import functools

import jax
import jax.numpy as jnp
from jax import lax
from jax.experimental import pallas as pl
from jax.experimental.pallas import tpu as pltpu
from jax.experimental.pallas import tpu_sc as plsc

N = 10000
E = 320000
D = 128
H = 32
AVG_DEG = 32.0

NC = 2
NS = 16
L = 16
NW = NC * NS
EPT = E // NW
EPT2 = 10240
E2 = NW * EPT2
C = 64
NCH = EPT2 // C
CG = 8
NG = NCH // CG
NP = 10240
RPT = NP // NS

f32 = jnp.float32
i32 = jnp.int32

_mesh = plsc.VectorSubcoreMesh(
    core_axis_name="c", subcore_axis_name="s", num_cores=NC, num_subcores=NS)

_sc_params = pltpu.CompilerParams(needs_layout_passes=False)


@functools.partial(
    pl.kernel,
    out_type=jax.ShapeDtypeStruct((NW, NCH, C), f32),
    mesh=_mesh,
    scratch_types=[
        pltpu.VMEM((NCH, C), i32),
        pltpu.VMEM((NCH, C), i32),
        pltpu.VMEM((3 * N,), f32),
        pltpu.VMEM((NCH, C), f32),
    ],
    compiler_params=_sc_params,
)
def _sc_r2(pos_hbm, src_hbm, dst_hbm, r2_hbm, si_v, di_v, pos_v, r2_v):
    c = lax.axis_index("c")
    s = lax.axis_index("s")
    wid = c * NS + s
    pltpu.sync_copy(pos_hbm, pos_v)
    pltpu.sync_copy(src_hbm.at[wid], si_v)
    pltpu.sync_copy(dst_hbm.at[wid], di_v)

    @pl.loop(0, NCH)
    def _chunk(j):
        @pl.loop(0, C // L)
        def _group(g):
            sids = si_v[j, pl.ds(g * L, L)] * 3
            dids = jnp.minimum(di_v[j, pl.ds(g * L, L)], N - 1) * 3
            dx = plsc.load_gather(pos_v, [dids]) - plsc.load_gather(pos_v, [sids])
            dy = (plsc.load_gather(pos_v, [dids + 1])
                  - plsc.load_gather(pos_v, [sids + 1]))
            dz = (plsc.load_gather(pos_v, [dids + 2])
                  - plsc.load_gather(pos_v, [sids + 2]))
            r2_v[j, pl.ds(g * L, L)] = dx * dx + dy * dy + dz * dz

    pltpu.sync_copy(r2_v, r2_hbm.at[wid])


@functools.partial(
    pl.kernel,
    out_type=jax.ShapeDtypeStruct((NC * NP, D), f32),
    mesh=_mesh,
    scratch_types=[
        pltpu.VMEM((CG, C), i32),
        pltpu.VMEM((CG, C), i32),
        pltpu.VMEM((2, C, D), f32),
        pltpu.VMEM((2, C, D), f32),
        pltpu.VMEM_SHARED((NP, D), f32),
        pltpu.SemaphoreType.DMA((2,)),
        pltpu.SemaphoreType.DMA((2,)),
        pltpu.SemaphoreType.DMA((2,)),
    ],
    compiler_params=_sc_params,
)
def _sc_pass(tf_hbm, rad_hbm, src_hbm, dst_hbm, zeros_hbm, out_hbm,
             si_v, di_v, g_v, r_v, acc_sh, gsem, rsem, ssem):
    c = lax.axis_index("c")
    s = lax.axis_index("s")
    wid = c * NS + s
    row0 = s * RPT
    pltpu.sync_copy(zeros_hbm.at[pl.ds(row0, RPT)], acc_sh.at[pl.ds(row0, RPT)])
    plsc.subcore_barrier()

    def _issue(jj, b, chunk):
        pltpu.async_copy(tf_hbm.at[si_v.at[jj]], g_v.at[b], gsem.at[b])
        pltpu.async_copy(rad_hbm.at[pl.ds(chunk * C, C)], r_v.at[b],
                         rsem.at[b])

    def _wait_scatter(b):
        pltpu.make_async_copy(g_v.at[b], acc_sh.at[di_v.at[0]],
                              ssem.at[b]).wait()

    pltpu.sync_copy(src_hbm.at[wid, pl.ds(0, CG)], si_v)
    pltpu.sync_copy(dst_hbm.at[wid, pl.ds(0, CG)], di_v)
    _issue(0, 0, wid * NCH)

    @pl.loop(0, NG)
    def _grp(g):
        for jj in range(CG):
            b = jj % 2
            chunk = wid * NCH + g * CG + jj
            pltpu.make_async_copy(tf_hbm.at[si_v.at[jj]], g_v.at[b],
                                  gsem.at[b]).wait()
            pltpu.make_async_copy(rad_hbm.at[pl.ds(chunk * C, C)],
                                  r_v.at[b], rsem.at[b]).wait()
            if jj < CG - 1:
                if jj + 1 >= 2:
                    _wait_scatter(1 - b)
                _issue(jj + 1, 1 - b, chunk + 1)

            @pl.loop(0, C)
            def _row(i):
                for k in range(D // L):
                    sl = (b, i, pl.ds(k * L, L))
                    g_v[sl] = g_v[sl] * r_v[sl]

            pltpu.async_copy(g_v.at[b], acc_sh.at[di_v.at[jj]], ssem.at[b],
                             add=True)

        _wait_scatter(0)
        _wait_scatter(1)

        @pl.when(g < NG - 1)
        def _next_group():
            pltpu.sync_copy(src_hbm.at[wid, pl.ds((g + 1) * CG, CG)], si_v)
            pltpu.sync_copy(dst_hbm.at[wid, pl.ds((g + 1) * CG, CG)], di_v)
            _issue(0, 0, wid * NCH + (g + 1) * CG)

    plsc.subcore_barrier()
    pltpu.sync_copy(acc_sh.at[pl.ds(row0, RPT)],
                    out_hbm.at[pl.ds(c * NP + row0, RPT)])


def _mm(x, w):
    def body(x_ref, w_ref, o_ref):
        o_ref[...] = jnp.dot(x_ref[...], w_ref[...], preferred_element_type=f32)

    return pl.pallas_call(
        body,
        out_shape=jax.ShapeDtypeStruct((x.shape[0], w.shape[1]), f32),
    )(x, w)


def _radial(r2col, w1l, b1l, w2l, b2l, w3l, b3l):
    B = 2048

    def body(r2_ref, w1_ref, b1_ref, w2_ref, b2_ref, w3_ref, b3_ref, o_ref):
        r = jnp.sqrt(r2_ref[...] + 1e-8)
        h = jnp.maximum(r * w1_ref[...] + b1_ref[...], 0.0)
        h = jnp.maximum(
            jnp.dot(h, w2_ref[...], preferred_element_type=f32) + b2_ref[...],
            0.0)
        o_ref[...] = jnp.dot(h, w3_ref[...], preferred_element_type=f32) + b3_ref[...]

    return pl.pallas_call(
        body,
        grid=(E2 // B,),
        in_specs=[
            pl.BlockSpec((B, 1), lambda i: (i, 0)),
            pl.BlockSpec((1, H), lambda i: (0, 0)),
            pl.BlockSpec((1, H), lambda i: (0, 0)),
            pl.BlockSpec((H, H), lambda i: (0, 0)),
            pl.BlockSpec((1, H), lambda i: (0, 0)),
            pl.BlockSpec((H, D), lambda i: (0, 0)),
            pl.BlockSpec((1, D), lambda i: (0, 0)),
        ],
        out_specs=pl.BlockSpec((B, D), lambda i: (i, 0)),
        out_shape=jax.ShapeDtypeStruct((E2, D), f32),
    )(r2col, w1l, b1l.reshape(1, H), w2l, b2l.reshape(1, H), w3l,
      b3l.reshape(1, D))


def _combine_mm(p, w):
    def body(p_ref, w_ref, o_ref, t_ref):
        out = jnp.maximum(
            (p_ref[:N, :] + p_ref[NP:NP + N, :]) * (1.0 / AVG_DEG), 0.0)
        o_ref[...] = out
        t_ref[...] = jnp.dot(out, w_ref[...], preferred_element_type=f32)

    return pl.pallas_call(
        body,
        out_shape=(jax.ShapeDtypeStruct((N, D), f32),
                   jax.ShapeDtypeStruct((N, D), f32)),
    )(p, w)


def _combine_skip_mm(p, skip, w):
    def body(p_ref, s_ref, w_ref, o_ref, t_ref):
        out = (jnp.maximum(
            (p_ref[:N, :] + p_ref[NP:NP + N, :]) * (1.0 / AVG_DEG), 0.0)
               + s_ref[...])
        o_ref[...] = out
        t_ref[...] = jnp.dot(out, w_ref[...], preferred_element_type=f32)

    return pl.pallas_call(
        body,
        out_shape=(jax.ShapeDtypeStruct((N, D), f32),
                   jax.ShapeDtypeStruct((N, D), f32)),
    )(p, skip, w)


def _final(p):
    def body(p_ref, o_ref):
        o_ref[...] = (p_ref[:N, :] + p_ref[NP:NP + N, :]) * (1.0 / AVG_DEG)

    return pl.pallas_call(
        body,
        out_shape=jax.ShapeDtypeStruct((N, D), f32),
    )(p)


def kernel(feat, pos, edge_index, W, w1, b1, w2, b2, w3, b3):
    pad = EPT2 - EPT
    src3 = jnp.pad(edge_index[0].reshape(NW, EPT), ((0, 0), (0, pad)),
                   constant_values=0).reshape(NW, NCH, C)
    dst3 = jnp.pad(edge_index[1].reshape(NW, EPT), ((0, 0), (0, pad)),
                   constant_values=N).reshape(NW, NCH, C)
    pos_flat = pos.reshape(3 * N)
    zeros = jnp.zeros((NP, D), f32)

    r2 = _sc_r2(pos_flat, src3, dst3).reshape(E2, 1)
    rads = [_radial(r2, w1[l], b1[l], w2[l], b2[l], w3[l], b3[l])
            for l in range(3)]

    tf = _mm(feat, W[0])
    p = _sc_pass(tf, rads[0], src3, dst3, zeros)
    out1, tf = _combine_mm(p, W[1])
    p = _sc_pass(tf, rads[1], src3, dst3, zeros)
    out2, tf = _combine_skip_mm(p, out1, W[1])
    p = _sc_pass(tf, rads[1], src3, dst3, zeros)
    out3, tf = _combine_skip_mm(p, out2, W[2])
    p = _sc_pass(tf, rads[2], src3, dst3, zeros)
    return _final(p)


# speedup vs baseline: 1.0239x; 1.0239x over previous
import functools

import jax
import jax.numpy as jnp
from jax import lax
from jax.experimental import pallas as pl
from jax.experimental.pallas import tpu as pltpu
from jax.experimental.pallas import tpu_sc as plsc

N = 10000
E = 320000
D = 128
H = 32
AVG_DEG = 32.0

NC = 2
NS = 16
L = 16
NW = NC * NS
EPT = E // NW
EPT2 = 10240
E2 = NW * EPT2
C = 64
NCH = EPT2 // C
CG = 16
NG = NCH // CG
NB = 2
NP = 10240
RPT = NP // NS

f32 = jnp.float32
i32 = jnp.int32

_mesh = plsc.VectorSubcoreMesh(
    core_axis_name="c", subcore_axis_name="s", num_cores=NC, num_subcores=NS)

_sc_params = pltpu.CompilerParams(needs_layout_passes=False)


@functools.partial(
    pl.kernel,
    out_type=jax.ShapeDtypeStruct((NW, NCH, C), f32),
    mesh=_mesh,
    scratch_types=[
        pltpu.VMEM((NCH, C), i32),
        pltpu.VMEM((NCH, C), i32),
        pltpu.VMEM((3 * N,), f32),
        pltpu.VMEM((NCH, C), f32),
    ],
    compiler_params=_sc_params,
)
def _sc_r2(pos_hbm, src_hbm, dst_hbm, r2_hbm, si_v, di_v, pos_v, r2_v):
    c = lax.axis_index("c")
    s = lax.axis_index("s")
    wid = c * NS + s
    pltpu.sync_copy(pos_hbm, pos_v)
    pltpu.sync_copy(src_hbm.at[wid], si_v)
    pltpu.sync_copy(dst_hbm.at[wid], di_v)

    @pl.loop(0, NCH)
    def _chunk(j):
        @pl.loop(0, C // L)
        def _group(g):
            sids = si_v[j, pl.ds(g * L, L)] * 3
            dids = jnp.minimum(di_v[j, pl.ds(g * L, L)], N - 1) * 3
            dx = plsc.load_gather(pos_v, [dids]) - plsc.load_gather(pos_v, [sids])
            dy = (plsc.load_gather(pos_v, [dids + 1])
                  - plsc.load_gather(pos_v, [sids + 1]))
            dz = (plsc.load_gather(pos_v, [dids + 2])
                  - plsc.load_gather(pos_v, [sids + 2]))
            r2_v[j, pl.ds(g * L, L)] = dx * dx + dy * dy + dz * dz

    pltpu.sync_copy(r2_v, r2_hbm.at[wid])


@functools.partial(
    pl.kernel,
    out_type=jax.ShapeDtypeStruct((NC * NP, D), f32),
    mesh=_mesh,
    scratch_types=[
        pltpu.VMEM((CG, C), i32),
        pltpu.VMEM((CG, C), i32),
        pltpu.VMEM((NB, C, D), f32),
        pltpu.VMEM((NB, C, D), f32),
        pltpu.VMEM_SHARED((NP, D), f32),
        pltpu.SemaphoreType.DMA((NB,)),
        pltpu.SemaphoreType.DMA((NB,)),
        pltpu.SemaphoreType.DMA((NB,)),
    ],
    compiler_params=_sc_params,
)
def _sc_pass(tf_hbm, rad_hbm, src_hbm, dst_hbm, zeros_hbm, out_hbm,
             si_v, di_v, g_v, r_v, acc_sh, gsem, rsem, ssem):
    c = lax.axis_index("c")
    s = lax.axis_index("s")
    wid = c * NS + s
    row0 = s * RPT
    pltpu.sync_copy(zeros_hbm.at[pl.ds(row0, RPT)], acc_sh.at[pl.ds(row0, RPT)])
    plsc.subcore_barrier()

    def _issue(jj, b, chunk):
        pltpu.async_copy(tf_hbm.at[si_v.at[jj]], g_v.at[b], gsem.at[b])
        pltpu.async_copy(rad_hbm.at[pl.ds(chunk * C, C)], r_v.at[b],
                         rsem.at[b])

    def _wait_scatter(b):
        pltpu.make_async_copy(g_v.at[b], acc_sh.at[di_v.at[0]],
                              ssem.at[b]).wait()

    pltpu.sync_copy(src_hbm.at[wid, pl.ds(0, CG)], si_v)
    pltpu.sync_copy(dst_hbm.at[wid, pl.ds(0, CG)], di_v)
    _issue(0, 0, wid * NCH)

    @pl.loop(0, NG)
    def _grp(g):
        for jj in range(CG):
            b = jj % NB
            chunk = wid * NCH + g * CG + jj
            pltpu.make_async_copy(tf_hbm.at[si_v.at[jj]], g_v.at[b],
                                  gsem.at[b]).wait()
            pltpu.make_async_copy(rad_hbm.at[pl.ds(chunk * C, C)],
                                  r_v.at[b], rsem.at[b]).wait()
            if jj < CG - 1:
                nb = (jj + 1) % NB
                if jj + 1 >= 2:
                    _wait_scatter(nb)
                _issue(jj + 1, nb, chunk + 1)

            @pl.loop(0, C)
            def _row(i):
                for k in range(D // L):
                    sl = (b, i, pl.ds(k * L, L))
                    g_v[sl] = g_v[sl] * r_v[sl]

            pltpu.async_copy(g_v.at[b], acc_sh.at[di_v.at[jj]], ssem.at[b],
                             add=True)

        for b in range(NB):
            _wait_scatter(b)

        @pl.when(g < NG - 1)
        def _next_group():
            pltpu.sync_copy(src_hbm.at[wid, pl.ds((g + 1) * CG, CG)], si_v)
            pltpu.sync_copy(dst_hbm.at[wid, pl.ds((g + 1) * CG, CG)], di_v)
            _issue(0, 0, wid * NCH + (g + 1) * CG)

    plsc.subcore_barrier()
    pltpu.sync_copy(acc_sh.at[pl.ds(row0, RPT)],
                    out_hbm.at[pl.ds(c * NP + row0, RPT)])


def _mm(x, w):
    def body(x_ref, w_ref, o_ref):
        o_ref[...] = jnp.dot(x_ref[...], w_ref[...], preferred_element_type=f32)

    return pl.pallas_call(
        body,
        out_shape=jax.ShapeDtypeStruct((x.shape[0], w.shape[1]), f32),
    )(x, w)


def _radial(r2col, w1l, b1l, w2l, b2l, w3l, b3l):
    B = 2048

    def body(r2_ref, w1_ref, b1_ref, w2_ref, b2_ref, w3_ref, b3_ref, o_ref):
        r = jnp.sqrt(r2_ref[...] + 1e-8)
        h = jnp.maximum(r * w1_ref[...] + b1_ref[...], 0.0)
        h = jnp.maximum(
            jnp.dot(h, w2_ref[...], preferred_element_type=f32) + b2_ref[...],
            0.0)
        o_ref[...] = jnp.dot(h, w3_ref[...], preferred_element_type=f32) + b3_ref[...]

    return pl.pallas_call(
        body,
        grid=(E2 // B,),
        in_specs=[
            pl.BlockSpec((B, 1), lambda i: (i, 0)),
            pl.BlockSpec((1, H), lambda i: (0, 0)),
            pl.BlockSpec((1, H), lambda i: (0, 0)),
            pl.BlockSpec((H, H), lambda i: (0, 0)),
            pl.BlockSpec((1, H), lambda i: (0, 0)),
            pl.BlockSpec((H, D), lambda i: (0, 0)),
            pl.BlockSpec((1, D), lambda i: (0, 0)),
        ],
        out_specs=pl.BlockSpec((B, D), lambda i: (i, 0)),
        out_shape=jax.ShapeDtypeStruct((E2, D), f32),
    )(r2col, w1l, b1l.reshape(1, H), w2l, b2l.reshape(1, H), w3l,
      b3l.reshape(1, D))


def _combine_mm(p, w):
    def body(p_ref, w_ref, o_ref, t_ref):
        out = jnp.maximum(
            (p_ref[:N, :] + p_ref[NP:NP + N, :]) * (1.0 / AVG_DEG), 0.0)
        o_ref[...] = out
        t_ref[...] = jnp.dot(out, w_ref[...], preferred_element_type=f32)

    return pl.pallas_call(
        body,
        out_shape=(jax.ShapeDtypeStruct((N, D), f32),
                   jax.ShapeDtypeStruct((N, D), f32)),
    )(p, w)


def _combine_skip_mm(p, skip, w):
    def body(p_ref, s_ref, w_ref, o_ref, t_ref):
        out = (jnp.maximum(
            (p_ref[:N, :] + p_ref[NP:NP + N, :]) * (1.0 / AVG_DEG), 0.0)
               + s_ref[...])
        o_ref[...] = out
        t_ref[...] = jnp.dot(out, w_ref[...], preferred_element_type=f32)

    return pl.pallas_call(
        body,
        out_shape=(jax.ShapeDtypeStruct((N, D), f32),
                   jax.ShapeDtypeStruct((N, D), f32)),
    )(p, skip, w)


def _final(p):
    def body(p_ref, o_ref):
        o_ref[...] = (p_ref[:N, :] + p_ref[NP:NP + N, :]) * (1.0 / AVG_DEG)

    return pl.pallas_call(
        body,
        out_shape=jax.ShapeDtypeStruct((N, D), f32),
    )(p)


def kernel(feat, pos, edge_index, W, w1, b1, w2, b2, w3, b3):
    pad = EPT2 - EPT
    src3 = jnp.pad(edge_index[0].reshape(NW, EPT), ((0, 0), (0, pad)),
                   constant_values=0).reshape(NW, NCH, C)
    dst3 = jnp.pad(edge_index[1].reshape(NW, EPT), ((0, 0), (0, pad)),
                   constant_values=N).reshape(NW, NCH, C)
    pos_flat = pos.reshape(3 * N)
    zeros = jnp.zeros((NP, D), f32)

    r2 = _sc_r2(pos_flat, src3, dst3).reshape(E2, 1)
    rads = [_radial(r2, w1[l], b1[l], w2[l], b2[l], w3[l], b3[l])
            for l in range(3)]

    tf = _mm(feat, W[0])
    p = _sc_pass(tf, rads[0], src3, dst3, zeros)
    out1, tf = _combine_mm(p, W[1])
    p = _sc_pass(tf, rads[1], src3, dst3, zeros)
    out2, tf = _combine_skip_mm(p, out1, W[1])
    p = _sc_pass(tf, rads[1], src3, dst3, zeros)
    out3, tf = _combine_skip_mm(p, out2, W[2])
    p = _sc_pass(tf, rads[2], src3, dst3, zeros)
    return _final(p)
